# baseline (speedup 1.0000x reference)
import functools

import jax
import jax.numpy as jnp
from jax import lax
from jax.experimental import pallas as pl
from jax.experimental.pallas import tpu as pltpu
from jax.experimental.pallas import tpu_sc as plsc

NC = 2
NS = 16
L = 16
NW = NC * NS
BATCH = 16384
D = 16
V = 1000000
UPG = 1024
NGRP = (V + UPG - 1) // UPG
NROW = NGRP * 128
GPW = 31
TAIL = V - 64
BPW = BATCH // NW
CHUNK = 128
NCHUNK = BPW // CHUNK
GROUPS = CHUNK // L


def _relayout_body(ue_hbm, ie_hbm, tu_hbm, ti_hbm, pu_hbm, pi_hbm,
                   vb_u, vb_i, pb_u, pb_i):
    wid = lax.axis_index("s") * NC + lax.axis_index("c")
    lanes = lax.iota(jnp.int32, L)

    def one_group(t, _):
        gid = wid * GPW + t

        @pl.when(gid < NGRP - 1)
        def _stage_full():
            start = pl.multiple_of(gid * UPG, UPG)
            pltpu.sync_copy(ue_hbm.at[:, pl.ds(start, UPG)], vb_u)
            pltpu.sync_copy(ie_hbm.at[:, pl.ds(start, UPG)], vb_i)

        @pl.when(gid == NGRP - 1)
        def _stage_last():
            pltpu.sync_copy(ue_hbm.at[:, pl.ds((NGRP - 1) * UPG, 512)],
                            vb_u.at[:, pl.ds(0, 512)])
            pltpu.sync_copy(ie_hbm.at[:, pl.ds((NGRP - 1) * UPG, 512)],
                            vb_i.at[:, pl.ds(0, 512)])
            pltpu.sync_copy(tu_hbm, vb_u.at[:, pl.ds(512, 128)])
            pltpu.sync_copy(ti_hbm, vb_i.at[:, pl.ds(512, 128)])

        @pl.when(gid < NGRP)
        def _do():

            def one_col(n, _):
                for b in range(UPG // 128):
                    col = jnp.full((L,), b * 128 + n, jnp.int32)
                    gu = plsc.load_gather(vb_u, [lanes, col])
                    gi = plsc.load_gather(vb_i, [lanes, col])
                    pb_u[n, pl.ds(b * L, L)] = gu
                    pb_i[n, pl.ds(b * L, L)] = gi
                return _

            lax.fori_loop(0, 128, one_col, None)
            pltpu.sync_copy(pb_u, pu_hbm.at[pl.ds(gid * 128, 128)])
            pltpu.sync_copy(pb_i, pi_hbm.at[pl.ds(gid * 128, 128)])

        return _

    lax.fori_loop(0, GPW, one_group, None)


def _gather_body(u_hbm, i_hbm, ue_hbm, ie_hbm, out_hbm,
                 u_idx, i_idx, ub_idx, ib_idx, u_blk, i_blk, out_v,
                 sem0, sem1):
    wid = lax.axis_index("s") * NC + lax.axis_index("c")

    pltpu.sync_copy(u_hbm.at[pl.ds(wid * NCHUNK, NCHUNK)], u_idx)
    pltpu.sync_copy(i_hbm.at[pl.ds(wid * NCHUNK, NCHUNK)], i_idx)

    for j in range(NCHUNK):
        for s in range(GROUPS):
            uv = u_idx[j, pl.ds(s * L, L)]
            iv = i_idx[j, pl.ds(s * L, L)]
            ub_idx[j, pl.ds(s * L, L)] = ((uv >> 10) << 7) | (uv & 127)
            ib_idx[j, pl.ds(s * L, L)] = ((iv >> 10) << 7) | (iv & 127)

    sems = [sem0, sem1]

    def fire(j):
        buf = j % 2
        return (
            pltpu.async_copy(ue_hbm.at[ub_idx.at[j]], u_blk.at[buf], sems[buf]),
            pltpu.async_copy(ie_hbm.at[ib_idx.at[j]], i_blk.at[buf], sems[buf]),
        )

    def lane_off(v):
        return ((v >> 7) & 7) << 4

    pending = fire(0)
    for j in range(NCHUNK):
        cu, ci = pending
        if j + 1 < NCHUNK:
            nxt = fire(j + 1)
        cu.wait()
        ci.wait()
        buf = j % 2
        lanes = lax.iota(jnp.int32, L)

        def group(s, _):
            uoffs = lane_off(u_idx[j, pl.ds(s * L, L)])
            ioffs = lane_off(i_idx[j, pl.ds(s * L, L)])
            acc = jnp.zeros((L,), jnp.float32)
            for k in range(L):
                ue = u_blk[buf, s * L + k, pl.ds(uoffs[k], L)]
                ie = i_blk[buf, s * L + k, pl.ds(ioffs[k], L)]
                acc = jnp.where(lanes == k, jnp.sum(ue * ie), acc)
            out_v[j, pl.ds(s * L, L)] = acc
            return _

        lax.fori_loop(0, GROUPS, group, None)
        if j + 1 < NCHUNK:
            pending = nxt

    pltpu.sync_copy(out_v, out_hbm.at[pl.ds(wid * NCHUNK, NCHUNK)])


@jax.jit
def kernel(u, i, user_emb, item_emb):
    mesh = plsc.VectorSubcoreMesh(core_axis_name="c", subcore_axis_name="s")
    relayout = pl.kernel(
        _relayout_body,
        out_type=[
            jax.ShapeDtypeStruct((NROW, 128), jnp.float32),
            jax.ShapeDtypeStruct((NROW, 128), jnp.float32),
        ],
        mesh=mesh,
        compiler_params=pltpu.CompilerParams(
            needs_layout_passes=False, use_tc_tiling_on_sc=True),
        scratch_types=[
            pltpu.VMEM((D, UPG), jnp.float32),
            pltpu.VMEM((D, UPG), jnp.float32),
            pltpu.VMEM((128, 128), jnp.float32),
            pltpu.VMEM((128, 128), jnp.float32),
        ],
    )
    gather = pl.kernel(
        _gather_body,
        out_type=jax.ShapeDtypeStruct((NW * NCHUNK, CHUNK), jnp.float32),
        mesh=mesh,
        compiler_params=pltpu.CompilerParams(
            needs_layout_passes=False, use_tc_tiling_on_sc=True),
        scratch_types=[
            pltpu.VMEM((NCHUNK, CHUNK), jnp.int32),
            pltpu.VMEM((NCHUNK, CHUNK), jnp.int32),
            pltpu.VMEM((NCHUNK, CHUNK), jnp.int32),
            pltpu.VMEM((NCHUNK, CHUNK), jnp.int32),
            pltpu.VMEM((2, CHUNK, 128), jnp.float32),
            pltpu.VMEM((2, CHUNK, 128), jnp.float32),
            pltpu.VMEM((NCHUNK, CHUNK), jnp.float32),
            pltpu.SemaphoreType.DMA,
            pltpu.SemaphoreType.DMA,
        ],
    )
    u2 = u.reshape(NW * NCHUNK, CHUNK)
    i2 = i.reshape(NW * NCHUNK, CHUNK)
    tail_u = jnp.pad(user_emb[TAIL:].T, ((0, 0), (0, 64)))
    tail_i = jnp.pad(item_emb[TAIL:].T, ((0, 0), (0, 64)))
    pu, pi = relayout(user_emb.T, item_emb.T, tail_u, tail_i)
    return gather(u2, i2, pu, pi).reshape(BATCH)


# speedup vs baseline: 1.7176x; 1.7176x over previous
import functools

import jax
import jax.numpy as jnp
from jax import lax
from jax.experimental import pallas as pl
from jax.experimental.pallas import tpu as pltpu
from jax.experimental.pallas import tpu_sc as plsc

NC = 2
NS = 16
L = 16
NW = NC * NS
BATCH = 16384
D = 16
V = 1000000
UPG = 1024
NGRP = (V + UPG - 1) // UPG
NROW = NGRP * 128
GPW = 31
TAIL = V - 64
BPW = BATCH // NW
CHUNK = 128
NCHUNK = BPW // CHUNK
GROUPS = CHUNK // L

TW = 8192
TGRID = (V + TW - 1) // TW


def _tc_relayout_body(ut_ref, uo_ref):
    x = ut_ref[...]
    for bg in range(TW // UPG):
        y = x[:, bg * UPG:(bg + 1) * UPG].T
        pieces = [y[b * 128:(b + 1) * 128, :] for b in range(8)]
        uo_ref[bg * 128:(bg + 1) * 128, :] = jnp.concatenate(pieces, axis=1)


def _tc_relayout(uet):
    return pl.pallas_call(
        _tc_relayout_body,
        grid=(TGRID,),
        in_specs=[pl.BlockSpec((D, TW), lambda g: (0, g))],
        out_specs=[pl.BlockSpec((TW // UPG * 128, 128), lambda g: (g, 0))],
        out_shape=[jax.ShapeDtypeStruct((NROW, 128), jnp.float32)],
    )(uet)[0]


def _sc_relayout_body(ie_hbm, ti_hbm, pi_hbm, vb_i, pb_i):
    wid = lax.axis_index("s") * NC + lax.axis_index("c")
    lanes = lax.iota(jnp.int32, L)

    def one_group(t, _):
        gid = wid * GPW + t

        @pl.when(gid < NGRP - 1)
        def _stage_full():
            start = pl.multiple_of(gid * UPG, UPG)
            pltpu.sync_copy(ie_hbm.at[:, pl.ds(start, UPG)], vb_i)

        @pl.when(gid == NGRP - 1)
        def _stage_last():
            pltpu.sync_copy(ie_hbm.at[:, pl.ds((NGRP - 1) * UPG, 512)],
                            vb_i.at[:, pl.ds(0, 512)])
            pltpu.sync_copy(ti_hbm, vb_i.at[:, pl.ds(512, 128)])

        @pl.when(gid < NGRP)
        def _do():

            def one_col(n, _):
                for b in range(UPG // 128):
                    col = jnp.full((L,), b * 128 + n, jnp.int32)
                    pb_i[n, pl.ds(b * L, L)] = plsc.load_gather(
                        vb_i, [lanes, col])
                return _

            lax.fori_loop(0, 128, one_col, None)
            pltpu.sync_copy(pb_i, pi_hbm.at[pl.ds(gid * 128, 128)])

        return _

    lax.fori_loop(0, GPW, one_group, None)


def _gather_body(u_hbm, i_hbm, ue_hbm, ie_hbm, out_hbm,
                 u_idx, i_idx, ub_idx, ib_idx, u_blk, i_blk, out_v,
                 sem0, sem1):
    wid = lax.axis_index("s") * NC + lax.axis_index("c")

    pltpu.sync_copy(u_hbm.at[pl.ds(wid * NCHUNK, NCHUNK)], u_idx)
    pltpu.sync_copy(i_hbm.at[pl.ds(wid * NCHUNK, NCHUNK)], i_idx)

    for j in range(NCHUNK):
        for s in range(GROUPS):
            uv = u_idx[j, pl.ds(s * L, L)]
            iv = i_idx[j, pl.ds(s * L, L)]
            ub_idx[j, pl.ds(s * L, L)] = ((uv >> 10) << 7) | (uv & 127)
            ib_idx[j, pl.ds(s * L, L)] = ((iv >> 10) << 7) | (iv & 127)

    sems = [sem0, sem1]

    def fire(j):
        buf = j % 2
        return (
            pltpu.async_copy(ue_hbm.at[ub_idx.at[j]], u_blk.at[buf], sems[buf]),
            pltpu.async_copy(ie_hbm.at[ib_idx.at[j]], i_blk.at[buf], sems[buf]),
        )

    pending = fire(0)
    for j in range(NCHUNK):
        cu, ci = pending
        if j + 1 < NCHUNK:
            nxt = fire(j + 1)
        cu.wait()
        ci.wait()
        buf = j % 2
        lanes = lax.iota(jnp.int32, L)

        def group(s, _):
            uoffs = ((u_idx[j, pl.ds(s * L, L)] >> 7) & 7) << 4
            ioffs = ((i_idx[j, pl.ds(s * L, L)] >> 7) & 7) << 4
            acc = jnp.zeros((L,), jnp.float32)
            for k in range(L):
                ue = u_blk[buf, s * L + k, pl.ds(uoffs[k], L)]
                ie = i_blk[buf, s * L + k, pl.ds(ioffs[k], L)]
                acc = jnp.where(lanes == k, jnp.sum(ue * ie), acc)
            out_v[j, pl.ds(s * L, L)] = acc
            return _

        lax.fori_loop(0, GROUPS, group, None)
        if j + 1 < NCHUNK:
            pending = nxt

    pltpu.sync_copy(out_v, out_hbm.at[pl.ds(wid * NCHUNK, NCHUNK)])


@jax.jit
def kernel(u, i, user_emb, item_emb):
    mesh = plsc.VectorSubcoreMesh(core_axis_name="c", subcore_axis_name="s")
    sc_relayout = pl.kernel(
        _sc_relayout_body,
        out_type=jax.ShapeDtypeStruct((NROW, 128), jnp.float32),
        mesh=mesh,
        compiler_params=pltpu.CompilerParams(
            needs_layout_passes=False, use_tc_tiling_on_sc=True),
        scratch_types=[
            pltpu.VMEM((D, UPG), jnp.float32),
            pltpu.VMEM((128, 128), jnp.float32),
        ],
    )
    gather = pl.kernel(
        _gather_body,
        out_type=jax.ShapeDtypeStruct((NW * NCHUNK, CHUNK), jnp.float32),
        mesh=mesh,
        compiler_params=pltpu.CompilerParams(
            needs_layout_passes=False, use_tc_tiling_on_sc=True),
        scratch_types=[
            pltpu.VMEM((NCHUNK, CHUNK), jnp.int32),
            pltpu.VMEM((NCHUNK, CHUNK), jnp.int32),
            pltpu.VMEM((NCHUNK, CHUNK), jnp.int32),
            pltpu.VMEM((NCHUNK, CHUNK), jnp.int32),
            pltpu.VMEM((2, CHUNK, 128), jnp.float32),
            pltpu.VMEM((2, CHUNK, 128), jnp.float32),
            pltpu.VMEM((NCHUNK, CHUNK), jnp.float32),
            pltpu.SemaphoreType.DMA,
            pltpu.SemaphoreType.DMA,
        ],
    )
    u2 = u.reshape(NW * NCHUNK, CHUNK)
    i2 = i.reshape(NW * NCHUNK, CHUNK)
    tail_i = jnp.pad(item_emb[TAIL:].T, ((0, 0), (0, 64)))
    pi = sc_relayout(item_emb.T, tail_i)
    pu = _tc_relayout(user_emb.T)
    return gather(u2, i2, pu, pi).reshape(BATCH)


# speedup vs baseline: 1.9664x; 1.1448x over previous
import functools

import jax
import jax.numpy as jnp
from jax import lax
from jax.experimental import pallas as pl
from jax.experimental.pallas import tpu as pltpu
from jax.experimental.pallas import tpu_sc as plsc

NC = 2
NS = 16
L = 16
NW = NC * NS
BATCH = 16384
D = 16
V = 1000000
UPG = 1024
NGRP = (V + UPG - 1) // UPG
NROW = NGRP * 128
GPW = 31
TAILG = (NGRP - 1) * UPG
BPW = BATCH // NW
CHUNK = 128
NCHUNK = BPW // CHUNK
GROUPS = CHUNK // L

TW = 8192
TGRID = (V + TW - 1) // TW


def _tc_relayout_body(ut_ref, uo_ref):
    x = ut_ref[...]
    for bg in range(TW // UPG):
        y = x[:, bg * UPG:(bg + 1) * UPG].T
        pieces = [y[b * 128:(b + 1) * 128, :] for b in range(8)]
        uo_ref[bg * 128:(bg + 1) * 128, :] = jnp.concatenate(pieces, axis=1)


def _tc_relayout(uet):
    return pl.pallas_call(
        _tc_relayout_body,
        grid=(TGRID,),
        in_specs=[pl.BlockSpec((D, TW), lambda g: (0, g))],
        out_specs=[pl.BlockSpec((TW // UPG * 128, 128), lambda g: (g, 0))],
        out_shape=[jax.ShapeDtypeStruct((NROW, 128), jnp.float32)],
    )(uet)[0]


def _sc_relayout_body(ie_hbm, ti_hbm, pi_hbm, vb, pb, sem_in, sem_out):
    wid = lax.axis_index("s") * NC + lax.axis_index("c")
    lanes = lax.iota(jnp.int32, L)

    def gid_of(t):
        return jnp.minimum(wid * GPW + t, NGRP - 1)

    def fire_in(t, slot):
        gid = gid_of(t)

        @pl.when(gid < NGRP - 1)
        def _full():
            start = pl.multiple_of(gid * UPG, UPG)
            pltpu.async_copy(ie_hbm.at[:, pl.ds(start, UPG)],
                             vb.at[slot], sem_in)

        @pl.when(gid == NGRP - 1)
        def _last():
            pltpu.async_copy(ti_hbm, vb.at[slot], sem_in)

    def drain_in(slot):
        pltpu.make_async_copy(
            ie_hbm.at[:, pl.ds(0, UPG)], vb.at[slot], sem_in).wait()

    def drain_out(slot):
        pltpu.make_async_copy(
            pb.at[slot], pi_hbm.at[pl.ds(0, 128)], sem_out).wait()

    fire_in(0, 0)
    for t in range(GPW):
        slot = t & 1
        if t + 1 < GPW:
            fire_in(t + 1, (t + 1) & 1)
        drain_in(slot)
        if t >= 2:
            drain_out(slot)

        def one_col(n, _):
            for b in range(UPG // 128):
                col = jnp.full((L,), b * 128 + n, jnp.int32)
                pb[slot, n, pl.ds(b * L, L)] = plsc.load_gather(
                    vb.at[slot], [lanes, col])
            return _

        lax.fori_loop(0, 128, one_col, None)
        row0 = pl.multiple_of(gid_of(t) * 128, 128)
        pltpu.async_copy(pb.at[slot], pi_hbm.at[pl.ds(row0, 128)], sem_out)

    drain_out((GPW - 2) & 1)
    drain_out((GPW - 1) & 1)


def _gather_body(u_hbm, i_hbm, ue_hbm, ie_hbm, out_hbm,
                 u_idx, i_idx, ub_idx, ib_idx, u_blk, i_blk, out_v,
                 sem0, sem1):
    wid = lax.axis_index("s") * NC + lax.axis_index("c")

    pltpu.sync_copy(u_hbm.at[pl.ds(wid * NCHUNK, NCHUNK)], u_idx)
    pltpu.sync_copy(i_hbm.at[pl.ds(wid * NCHUNK, NCHUNK)], i_idx)

    for j in range(NCHUNK):
        for s in range(GROUPS):
            uv = u_idx[j, pl.ds(s * L, L)]
            iv = i_idx[j, pl.ds(s * L, L)]
            ub_idx[j, pl.ds(s * L, L)] = ((uv >> 10) << 7) | (uv & 127)
            ib_idx[j, pl.ds(s * L, L)] = ((iv >> 10) << 7) | (iv & 127)

    sems = [sem0, sem1]

    def fire(j):
        buf = j % 2
        return (
            pltpu.async_copy(ue_hbm.at[ub_idx.at[j]], u_blk.at[buf], sems[buf]),
            pltpu.async_copy(ie_hbm.at[ib_idx.at[j]], i_blk.at[buf], sems[buf]),
        )

    pending = fire(0)
    for j in range(NCHUNK):
        cu, ci = pending
        if j + 1 < NCHUNK:
            nxt = fire(j + 1)
        cu.wait()
        ci.wait()
        buf = j % 2
        lanes = lax.iota(jnp.int32, L)

        def group(s, _):
            uoffs = ((u_idx[j, pl.ds(s * L, L)] >> 7) & 7) << 4
            ioffs = ((i_idx[j, pl.ds(s * L, L)] >> 7) & 7) << 4
            acc = jnp.zeros((L,), jnp.float32)
            for k in range(L):
                ue = u_blk[buf, s * L + k, pl.ds(uoffs[k], L)]
                ie = i_blk[buf, s * L + k, pl.ds(ioffs[k], L)]
                acc = jnp.where(lanes == k, jnp.sum(ue * ie), acc)
            out_v[j, pl.ds(s * L, L)] = acc
            return _

        lax.fori_loop(0, GROUPS, group, None)
        if j + 1 < NCHUNK:
            pending = nxt

    pltpu.sync_copy(out_v, out_hbm.at[pl.ds(wid * NCHUNK, NCHUNK)])


@jax.jit
def kernel(u, i, user_emb, item_emb):
    mesh = plsc.VectorSubcoreMesh(core_axis_name="c", subcore_axis_name="s")
    sc_relayout = pl.kernel(
        _sc_relayout_body,
        out_type=jax.ShapeDtypeStruct((NROW, 128), jnp.float32),
        mesh=mesh,
        compiler_params=pltpu.CompilerParams(
            needs_layout_passes=False, use_tc_tiling_on_sc=True),
        scratch_types=[
            pltpu.VMEM((2, D, UPG), jnp.float32),
            pltpu.VMEM((2, 128, 128), jnp.float32),
            pltpu.SemaphoreType.DMA,
            pltpu.SemaphoreType.DMA,
        ],
    )
    gather = pl.kernel(
        _gather_body,
        out_type=jax.ShapeDtypeStruct((NW * NCHUNK, CHUNK), jnp.float32),
        mesh=mesh,
        compiler_params=pltpu.CompilerParams(
            needs_layout_passes=False, use_tc_tiling_on_sc=True),
        scratch_types=[
            pltpu.VMEM((NCHUNK, CHUNK), jnp.int32),
            pltpu.VMEM((NCHUNK, CHUNK), jnp.int32),
            pltpu.VMEM((NCHUNK, CHUNK), jnp.int32),
            pltpu.VMEM((NCHUNK, CHUNK), jnp.int32),
            pltpu.VMEM((2, CHUNK, 128), jnp.float32),
            pltpu.VMEM((2, CHUNK, 128), jnp.float32),
            pltpu.VMEM((NCHUNK, CHUNK), jnp.float32),
            pltpu.SemaphoreType.DMA,
            pltpu.SemaphoreType.DMA,
        ],
    )
    u2 = u.reshape(NW * NCHUNK, CHUNK)
    i2 = i.reshape(NW * NCHUNK, CHUNK)
    tail_i = jnp.pad(item_emb[TAILG:].T, ((0, 0), (0, UPG - (V - TAILG))))
    pi = sc_relayout(item_emb.T, tail_i)
    pu = _tc_relayout(user_emb.T)
    return gather(u2, i2, pu, pi).reshape(BATCH)


# speedup vs baseline: 2.8377x; 1.4431x over previous
import functools

import jax
import jax.numpy as jnp
from jax import lax
from jax.experimental import pallas as pl
from jax.experimental.pallas import tpu as pltpu
from jax.experimental.pallas import tpu_sc as plsc

NC = 2
NS = 16
L = 16
NW = NC * NS
BATCH = 16384
D = 16
V = 1000000
UPG = 1024
NGRP = (V + UPG - 1) // UPG
NROW = NGRP * 128
GPW = 31
TAILG = (NGRP - 1) * UPG
BPW = BATCH // NW
CHUNK = 128
NCHUNK = BPW // CHUNK
GROUPS = CHUNK // L

TW = 8192
TGRID = (V + TW - 1) // TW


def _tc_relayout_body(ut_ref, uo_ref):
    x = ut_ref[...]
    for bg in range(TW // UPG):
        y = x[:, bg * UPG:(bg + 1) * UPG].T
        pieces = [y[b * 128:(b + 1) * 128, :] for b in range(8)]
        uo_ref[bg * 128:(bg + 1) * 128, :] = jnp.concatenate(pieces, axis=1)


def _tc_relayout(uet):
    return pl.pallas_call(
        _tc_relayout_body,
        grid=(TGRID,),
        in_specs=[pl.BlockSpec((D, TW), lambda g: (0, g))],
        out_specs=[pl.BlockSpec((TW // UPG * 128, 128), lambda g: (g, 0))],
        out_shape=[jax.ShapeDtypeStruct((NROW, 128), jnp.float32)],
    )(uet)[0]


def _sc_relayout_body(ie_hbm, ti_hbm, pi_hbm, vb, pb, sem_in, sem_out):
    wid = lax.axis_index("s") * NC + lax.axis_index("c")
    lanes = lax.iota(jnp.int32, L)

    def gid_of(t):
        return jnp.minimum(wid * GPW + t, NGRP - 1)

    def fire_in(t, slot):
        gid = gid_of(t)

        @pl.when(gid < NGRP - 1)
        def _full():
            start = pl.multiple_of(gid * UPG, UPG)
            pltpu.async_copy(ie_hbm.at[:, pl.ds(start, UPG)],
                             vb.at[slot], sem_in)

        @pl.when(gid == NGRP - 1)
        def _last():
            pltpu.async_copy(ti_hbm, vb.at[slot], sem_in)

    def drain_in(slot):
        pltpu.make_async_copy(
            ie_hbm.at[:, pl.ds(0, UPG)], vb.at[slot], sem_in).wait()

    def drain_out(slot):
        pltpu.make_async_copy(
            pb.at[slot], pi_hbm.at[pl.ds(0, 128)], sem_out).wait()

    fire_in(0, 0)
    for t in range(GPW):
        slot = t & 1
        if t + 1 < GPW:
            fire_in(t + 1, (t + 1) & 1)
        drain_in(slot)
        if t >= 2:
            drain_out(slot)

        rsub = lanes >> 3
        csub = lanes & 7

        def one_pair(n2, _):
            rows = 2 * n2 + rsub
            for d in range(D):
                v = vb[slot, d, pl.ds(n2 * L, L)]
                plsc.store_scatter(pb.at[slot], [rows, d * 8 + csub], v)
            return _

        lax.fori_loop(0, 64, one_pair, None)
        row0 = pl.multiple_of(gid_of(t) * 128, 128)
        pltpu.async_copy(pb.at[slot], pi_hbm.at[pl.ds(row0, 128)], sem_out)

    drain_out((GPW - 2) & 1)
    drain_out((GPW - 1) & 1)


def _gather_body(u_hbm, i_hbm, ue_hbm, ie_hbm, out_hbm,
                 u_idx, i_idx, ub_idx, ib_idx, u_blk, i_blk, out_v,
                 sem0, sem1):
    wid = lax.axis_index("s") * NC + lax.axis_index("c")

    pltpu.sync_copy(u_hbm.at[pl.ds(wid * NCHUNK, NCHUNK)], u_idx)
    pltpu.sync_copy(i_hbm.at[pl.ds(wid * NCHUNK, NCHUNK)], i_idx)

    for j in range(NCHUNK):
        for s in range(GROUPS):
            uv = u_idx[j, pl.ds(s * L, L)]
            iv = i_idx[j, pl.ds(s * L, L)]
            ub_idx[j, pl.ds(s * L, L)] = ((uv >> 10) << 7) | (uv & 127)
            ib_idx[j, pl.ds(s * L, L)] = iv >> 3

    sems = [sem0, sem1]

    def fire(j):
        buf = j % 2
        return (
            pltpu.async_copy(ue_hbm.at[ub_idx.at[j]], u_blk.at[buf], sems[buf]),
            pltpu.async_copy(ie_hbm.at[ib_idx.at[j]], i_blk.at[buf], sems[buf]),
        )

    pending = fire(0)
    for j in range(NCHUNK):
        cu, ci = pending
        if j + 1 < NCHUNK:
            nxt = fire(j + 1)
        cu.wait()
        ci.wait()
        buf = j % 2
        lanes = lax.iota(jnp.int32, L)

        bufv = jnp.full((L,), buf, jnp.int32)

        def group(s, _):
            uoffs = ((u_idx[j, pl.ds(s * L, L)] >> 7) & 7) << 4
            ioffs = i_idx[j, pl.ds(s * L, L)] & 7
            acc = jnp.zeros((L,), jnp.float32)
            for k in range(L):
                ue = u_blk[buf, s * L + k, pl.ds(uoffs[k], L)]
                ie = plsc.load_gather(
                    i_blk, [bufv, jnp.full((L,), s * L + k, jnp.int32),
                            ioffs[k] + lanes * 8])
                acc = jnp.where(lanes == k, jnp.sum(ue * ie), acc)
            out_v[j, pl.ds(s * L, L)] = acc
            return _

        lax.fori_loop(0, GROUPS, group, None)
        if j + 1 < NCHUNK:
            pending = nxt

    pltpu.sync_copy(out_v, out_hbm.at[pl.ds(wid * NCHUNK, NCHUNK)])


@jax.jit
def kernel(u, i, user_emb, item_emb):
    mesh = plsc.VectorSubcoreMesh(core_axis_name="c", subcore_axis_name="s")
    sc_relayout = pl.kernel(
        _sc_relayout_body,
        out_type=jax.ShapeDtypeStruct((NROW, 128), jnp.float32),
        mesh=mesh,
        compiler_params=pltpu.CompilerParams(
            needs_layout_passes=False, use_tc_tiling_on_sc=True),
        scratch_types=[
            pltpu.VMEM((2, D, UPG), jnp.float32),
            pltpu.VMEM((2, 128, 128), jnp.float32),
            pltpu.SemaphoreType.DMA,
            pltpu.SemaphoreType.DMA,
        ],
    )
    gather = pl.kernel(
        _gather_body,
        out_type=jax.ShapeDtypeStruct((NW * NCHUNK, CHUNK), jnp.float32),
        mesh=mesh,
        compiler_params=pltpu.CompilerParams(
            needs_layout_passes=False, use_tc_tiling_on_sc=True),
        scratch_types=[
            pltpu.VMEM((NCHUNK, CHUNK), jnp.int32),
            pltpu.VMEM((NCHUNK, CHUNK), jnp.int32),
            pltpu.VMEM((NCHUNK, CHUNK), jnp.int32),
            pltpu.VMEM((NCHUNK, CHUNK), jnp.int32),
            pltpu.VMEM((2, CHUNK, 128), jnp.float32),
            pltpu.VMEM((2, CHUNK, 128), jnp.float32),
            pltpu.VMEM((NCHUNK, CHUNK), jnp.float32),
            pltpu.SemaphoreType.DMA,
            pltpu.SemaphoreType.DMA,
        ],
    )
    u2 = u.reshape(NW * NCHUNK, CHUNK)
    i2 = i.reshape(NW * NCHUNK, CHUNK)
    tail_i = jnp.pad(item_emb[TAILG:].T, ((0, 0), (0, UPG - (V - TAILG))))
    pi = sc_relayout(item_emb.T, tail_i)
    pu = _tc_relayout(user_emb.T)
    return gather(u2, i2, pu, pi).reshape(BATCH)


# speedup vs baseline: 2.9027x; 1.0229x over previous
import functools

import jax
import jax.numpy as jnp
from jax import lax
from jax.experimental import pallas as pl
from jax.experimental.pallas import tpu as pltpu
from jax.experimental.pallas import tpu_sc as plsc

NC = 2
NS = 16
L = 16
NW = NC * NS
BATCH = 16384
D = 16
V = 1000000
UPG = 1024
NGRP = (V + UPG - 1) // UPG
NROW = NGRP * 128
GPW = 31
TAILG = (NGRP - 1) * UPG
BPW = BATCH // NW
CHUNK = 128
NCHUNK = BPW // CHUNK
GROUPS = CHUNK // L

TW = 16384
TGRID = (V + TW - 1) // TW


def _tc_relayout_body(ut_ref, uo_ref):
    x = ut_ref[...]
    for bg in range(TW // UPG):
        y = x[:, bg * UPG:(bg + 1) * UPG].T
        pieces = [y[b * 128:(b + 1) * 128, :] for b in range(8)]
        uo_ref[bg * 128:(bg + 1) * 128, :] = jnp.concatenate(pieces, axis=1)


def _tc_relayout(uet):
    return pl.pallas_call(
        _tc_relayout_body,
        grid=(TGRID,),
        in_specs=[pl.BlockSpec((D, TW), lambda g: (0, g))],
        out_specs=[pl.BlockSpec((TW // UPG * 128, 128), lambda g: (g, 0))],
        out_shape=[jax.ShapeDtypeStruct((NROW, 128), jnp.float32)],
    )(uet)[0]


def _sc_relayout_body(ie_hbm, ti_hbm, pi_hbm, vb, pb, sem_in, sem_out):
    wid = lax.axis_index("s") * NC + lax.axis_index("c")
    lanes = lax.iota(jnp.int32, L)

    def gid_of(t):
        return jnp.minimum(wid * GPW + t, NGRP - 1)

    def fire_in(t, slot):
        gid = gid_of(t)

        @pl.when(gid < NGRP - 1)
        def _full():
            start = pl.multiple_of(gid * UPG, UPG)
            pltpu.async_copy(ie_hbm.at[:, pl.ds(start, UPG)],
                             vb.at[slot], sem_in)

        @pl.when(gid == NGRP - 1)
        def _last():
            pltpu.async_copy(ti_hbm, vb.at[slot], sem_in)

    def drain_in(slot):
        pltpu.make_async_copy(
            ie_hbm.at[:, pl.ds(0, UPG)], vb.at[slot], sem_in).wait()

    def drain_out(slot):
        pltpu.make_async_copy(
            pb.at[slot], pi_hbm.at[pl.ds(0, 128)], sem_out).wait()

    fire_in(0, 0)
    for t in range(GPW):
        slot = t & 1
        if t + 1 < GPW:
            fire_in(t + 1, (t + 1) & 1)
        drain_in(slot)
        if t >= 2:
            drain_out(slot)

        rsub = lanes >> 3
        csub = lanes & 7

        def one_pair(n2, _):
            rows = 2 * n2 + rsub
            for d in range(D):
                v = vb[slot, d, pl.ds(n2 * L, L)]
                plsc.store_scatter(pb.at[slot], [rows, d * 8 + csub], v)
            return _

        lax.fori_loop(0, 64, one_pair, None)
        row0 = pl.multiple_of(gid_of(t) * 128, 128)
        pltpu.async_copy(pb.at[slot], pi_hbm.at[pl.ds(row0, 128)], sem_out)

    drain_out((GPW - 2) & 1)
    drain_out((GPW - 1) & 1)


def _gather_body(u_hbm, i_hbm, ue_hbm, ie_hbm, out_hbm,
                 u_idx, i_idx, ub_idx, ib_idx, u_blk, i_blk, out_v,
                 sem0, sem1):
    wid = lax.axis_index("s") * NC + lax.axis_index("c")

    pltpu.sync_copy(u_hbm.at[pl.ds(wid * NCHUNK, NCHUNK)], u_idx)
    pltpu.sync_copy(i_hbm.at[pl.ds(wid * NCHUNK, NCHUNK)], i_idx)

    for j in range(NCHUNK):
        for s in range(GROUPS):
            uv = u_idx[j, pl.ds(s * L, L)]
            iv = i_idx[j, pl.ds(s * L, L)]
            ub_idx[j, pl.ds(s * L, L)] = ((uv >> 10) << 7) | (uv & 127)
            ib_idx[j, pl.ds(s * L, L)] = iv >> 3

    sems = [sem0, sem1]

    def fire(j):
        buf = j % 2
        return (
            pltpu.async_copy(ue_hbm.at[ub_idx.at[j]], u_blk.at[buf], sems[buf]),
            pltpu.async_copy(ie_hbm.at[ib_idx.at[j]], i_blk.at[buf], sems[buf]),
        )

    pending = fire(0)
    for j in range(NCHUNK):
        cu, ci = pending
        if j + 1 < NCHUNK:
            nxt = fire(j + 1)
        cu.wait()
        ci.wait()
        buf = j % 2
        lanes = lax.iota(jnp.int32, L)

        bufv = jnp.full((L,), buf, jnp.int32)

        def group(s, _):
            uoffs = ((u_idx[j, pl.ds(s * L, L)] >> 7) & 7) << 4
            ioffs = i_idx[j, pl.ds(s * L, L)] & 7
            acc = jnp.zeros((L,), jnp.float32)
            for k in range(L):
                ue = u_blk[buf, s * L + k, pl.ds(uoffs[k], L)]
                ie = plsc.load_gather(
                    i_blk, [bufv, jnp.full((L,), s * L + k, jnp.int32),
                            ioffs[k] + lanes * 8])
                acc = jnp.where(lanes == k, jnp.sum(ue * ie), acc)
            out_v[j, pl.ds(s * L, L)] = acc
            return _

        lax.fori_loop(0, GROUPS, group, None)
        if j + 1 < NCHUNK:
            pending = nxt

    pltpu.sync_copy(out_v, out_hbm.at[pl.ds(wid * NCHUNK, NCHUNK)])


@jax.jit
def kernel(u, i, user_emb, item_emb):
    mesh = plsc.VectorSubcoreMesh(core_axis_name="c", subcore_axis_name="s")
    sc_relayout = pl.kernel(
        _sc_relayout_body,
        out_type=jax.ShapeDtypeStruct((NROW, 128), jnp.float32),
        mesh=mesh,
        compiler_params=pltpu.CompilerParams(
            needs_layout_passes=False, use_tc_tiling_on_sc=True),
        scratch_types=[
            pltpu.VMEM((2, D, UPG), jnp.float32),
            pltpu.VMEM((2, 128, 128), jnp.float32),
            pltpu.SemaphoreType.DMA,
            pltpu.SemaphoreType.DMA,
        ],
    )
    gather = pl.kernel(
        _gather_body,
        out_type=jax.ShapeDtypeStruct((NW * NCHUNK, CHUNK), jnp.float32),
        mesh=mesh,
        compiler_params=pltpu.CompilerParams(
            needs_layout_passes=False, use_tc_tiling_on_sc=True),
        scratch_types=[
            pltpu.VMEM((NCHUNK, CHUNK), jnp.int32),
            pltpu.VMEM((NCHUNK, CHUNK), jnp.int32),
            pltpu.VMEM((NCHUNK, CHUNK), jnp.int32),
            pltpu.VMEM((NCHUNK, CHUNK), jnp.int32),
            pltpu.VMEM((2, CHUNK, 128), jnp.float32),
            pltpu.VMEM((2, CHUNK, 128), jnp.float32),
            pltpu.VMEM((NCHUNK, CHUNK), jnp.float32),
            pltpu.SemaphoreType.DMA,
            pltpu.SemaphoreType.DMA,
        ],
    )
    u2 = u.reshape(NW * NCHUNK, CHUNK)
    i2 = i.reshape(NW * NCHUNK, CHUNK)
    tail_i = jnp.pad(item_emb[TAILG:].T, ((0, 0), (0, UPG - (V - TAILG))))
    pi = sc_relayout(item_emb.T, tail_i)
    pu = _tc_relayout(user_emb.T)
    return gather(u2, i2, pu, pi).reshape(BATCH)
